# baseline (speedup 1.0000x reference)
import jax
import jax.numpy as jnp
from jax.experimental import pallas as pl
from jax.experimental.pallas import tpu as pltpu

N_GRID = 50000
N_MESH0 = 3125
N_MESH1 = 12500
HID = 128


def _mm_kernel(x_ref, w_ref, b_ref, o_ref):
    o_ref[...] = jnp.dot(x_ref[...], w_ref[...],
                         preferred_element_type=jnp.float32) + b_ref[...]


def _mm(x, w, b, block=1000):
    n = x.shape[0]
    return pl.pallas_call(
        _mm_kernel,
        grid=(n // block,),
        in_specs=[
            pl.BlockSpec((block, x.shape[1]), lambda i: (i, 0)),
            pl.BlockSpec((x.shape[1], w.shape[1]), lambda i: (0, 0)),
            pl.BlockSpec((w.shape[1],), lambda i: (0,)),
        ],
        out_specs=pl.BlockSpec((block, w.shape[1]), lambda i: (i, 0)),
        out_shape=jax.ShapeDtypeStruct((n, w.shape[1]), jnp.float32),
    )(x, w, b)


def _mp(p, x_src, x_dst, edge_index, e_feat, n_dst):
    src, dst = edge_index[0], edge_index[1]
    m = jnp.concatenate([x_src[src], x_dst[dst], e_feat], axis=-1)
    m = jax.nn.relu(m @ p['W_e'] + p['b_e'])
    agg = jax.ops.segment_sum(m, dst, num_segments=n_dst)
    u = jnp.concatenate([x_dst, agg], axis=-1) @ p['W_n'] + p['b_n']
    return x_dst + u


def kernel(grid_features, g2m_edge_index, g2m_features, m2g_edge_index, m2g_features,
           m2m_edge_index_0, m2m_features_0, m2m_edge_index_1, m2m_features_1,
           up_edge_index_0, up_features_0, down_edge_index_0, down_features_0,
           mesh_features_0, mesh_features_1, params):
    grid_rep = grid_features @ params['W_grid_enc'] + params['b_grid_enc']
    mesh0 = mesh_features_0 @ params['W_mesh_enc_0'] + params['b_mesh_enc_0']
    mesh1 = mesh_features_1 @ params['W_mesh_enc_1'] + params['b_mesh_enc_1']
    mesh1 = _mp(params['g2m'], grid_rep, mesh1, g2m_edge_index, g2m_features, N_MESH1)
    mesh1 = _mp(params['same_1'], mesh1, mesh1, m2m_edge_index_1, m2m_features_1, N_MESH1)
    mesh0 = _mp(params['up_0'], mesh1, mesh0, up_edge_index_0, up_features_0, N_MESH0)
    mesh0 = _mp(params['same_0'], mesh0, mesh0, m2m_edge_index_0, m2m_features_0, N_MESH0)
    mesh1 = _mp(params['down_0'], mesh0, mesh1, down_edge_index_0, down_features_0, N_MESH1)
    mesh1 = _mp(params['same_1'], mesh1, mesh1, m2m_edge_index_1, m2m_features_1, N_MESH1)
    grid_rep_out = _mp(params['m2g'], mesh1, grid_rep, m2g_edge_index, m2g_features, N_GRID)
    delta = _mm(grid_rep_out, params['W_grid_dec'], params['b_grid_dec'])
    return delta


# speedup vs baseline: 1.6278x; 1.6278x over previous
import functools

import jax
import jax.numpy as jnp
from jax import lax
from jax.experimental import pallas as pl
from jax.experimental.pallas import tpu as pltpu
from jax.experimental.pallas import tpu_sc as plsc

N_GRID = 50000
N_MESH0 = 3125
N_MESH1 = 12500
HID = 128

N0P = 3328
N1P = 12544

_NC = 2
_NS = 16
_L = 16
_NW = _NC * _NS


def _mm_bias_kernel(x_ref, w_ref, b_ref, o_ref):
    o_ref[...] = jnp.dot(x_ref[...], w_ref[...],
                         preferred_element_type=jnp.float32) + b_ref[...]


def _mm(x, w, b, blk):
    n, kdim = x.shape
    return pl.pallas_call(
        _mm_bias_kernel,
        grid=(n // blk,),
        in_specs=[
            pl.BlockSpec((blk, kdim), lambda i: (i, 0)),
            pl.BlockSpec((kdim, w.shape[1]), lambda i: (0, 0)),
            pl.BlockSpec((w.shape[1],), lambda i: (0,)),
        ],
        out_specs=pl.BlockSpec((blk, w.shape[1]), lambda i: (i, 0)),
        out_shape=jax.ShapeDtypeStruct((n, w.shape[1]), jnp.float32),
    )(x, w, b)


def _enc2_kernel(x_ref, w_ref, b_ref, o_ref):
    x = x_ref[...]
    w = w_ref[...]
    o_ref[...] = x[:, 0:1] * w[0:1, :] + x[:, 1:2] * w[1:2, :] + b_ref[...]


def _enc2(x, w, b):
    n = x.shape[0]
    return pl.pallas_call(
        _enc2_kernel,
        grid=(1,),
        in_specs=[
            pl.BlockSpec((n, 2), lambda i: (0, 0)),
            pl.BlockSpec((2, HID), lambda i: (0, 0)),
            pl.BlockSpec((HID,), lambda i: (0,)),
        ],
        out_specs=pl.BlockSpec((n, HID), lambda i: (0, 0)),
        out_shape=jax.ShapeDtypeStruct((n, HID), jnp.float32),
    )(x, w, b)


def _c_kernel(ef_ref, w_ref, o_ref):
    ef = ef_ref[...]
    w = w_ref[...]
    o_ref[...] = (ef[:, 0:1] * w[0:1, :] + ef[:, 1:2] * w[1:2, :]
                  + ef[:, 2:3] * w[2:3, :] + ef[:, 3:4] * w[3:4, :])


def _edge_c(ef, w, blk=2048):
    n = ef.shape[0]
    return pl.pallas_call(
        _c_kernel,
        grid=(n // blk,),
        in_specs=[
            pl.BlockSpec((blk, 4), lambda i: (i, 0)),
            pl.BlockSpec((4, HID), lambda i: (0, 0)),
        ],
        out_specs=pl.BlockSpec((blk, HID), lambda i: (i, 0)),
        out_shape=jax.ShapeDtypeStruct((n, HID), jnp.float32),
    )(ef, w)


def _upd_kernel(x_ref, a0_ref, a1_ref, w1_ref, w2_ref, b_ref, o_ref):
    x = x_ref[...]
    agg = a0_ref[...] + a1_ref[...]
    o_ref[...] = (x + jnp.dot(x, w1_ref[...], preferred_element_type=jnp.float32)
                  + jnp.dot(agg, w2_ref[...], preferred_element_type=jnp.float32)
                  + b_ref[...])


def _upd(x, a0, a1, w1, w2, b, blk):
    n = x.shape[0]
    return pl.pallas_call(
        _upd_kernel,
        grid=(n // blk,),
        in_specs=[
            pl.BlockSpec((blk, HID), lambda i: (i, 0)),
            pl.BlockSpec((blk, HID), lambda i: (i, 0)),
            pl.BlockSpec((blk, HID), lambda i: (i, 0)),
            pl.BlockSpec((HID, HID), lambda i: (0, 0)),
            pl.BlockSpec((HID, HID), lambda i: (0, 0)),
            pl.BlockSpec((HID,), lambda i: (0,)),
        ],
        out_specs=pl.BlockSpec((blk, HID), lambda i: (i, 0)),
        out_shape=jax.ShapeDtypeStruct((n, HID), jnp.float32),
    )(x, a0, a1, w1, w2, b)


def _m2g_dec_kernel(x_ref, a0_ref, a1_ref, w1_ref, w2_ref, bn_ref,
                    wd_ref, bd_ref, o_ref):
    x = x_ref[...]
    agg = a0_ref[...] + a1_ref[...]
    t = (x + jnp.dot(x, w1_ref[...], preferred_element_type=jnp.float32)
         + jnp.dot(agg, w2_ref[...], preferred_element_type=jnp.float32)
         + bn_ref[...])
    o_ref[...] = jnp.dot(t, wd_ref[...],
                         preferred_element_type=jnp.float32) + bd_ref[...]


def _m2g_dec(x, a0, a1, w1, w2, bn, wd, bd, blk=1000):
    n = x.shape[0]
    return pl.pallas_call(
        _m2g_dec_kernel,
        grid=(n // blk,),
        in_specs=[
            pl.BlockSpec((blk, HID), lambda i: (i, 0)),
            pl.BlockSpec((blk, HID), lambda i: (i, 0)),
            pl.BlockSpec((blk, HID), lambda i: (i, 0)),
            pl.BlockSpec((HID, HID), lambda i: (0, 0)),
            pl.BlockSpec((HID, HID), lambda i: (0, 0)),
            pl.BlockSpec((HID,), lambda i: (0,)),
            pl.BlockSpec((HID, HID), lambda i: (0, 0)),
            pl.BlockSpec((HID,), lambda i: (0,)),
        ],
        out_specs=pl.BlockSpec((blk, HID), lambda i: (i, 0)),
        out_shape=jax.ShapeDtypeStruct((n, HID), jnp.float32),
    )(x, a0, a1, w1, w2, bn, wd, bd)


def _sc_edge_agg(a_tab, b_tab, c_tab, src_idx, dst_idx, n_dst_pad):
    e_pad = src_idx.shape[0]
    per_w = e_pad // _NW
    budget = 2_097_151 - n_dst_pad * HID
    kc = 128
    while kc > 8 and (kc * (3 * HID + 2) * _NS > budget or per_w % kc):
        kc //= 2
    n_chunks = per_w // kc
    rows_per_tile = n_dst_pad // _NS
    mesh = plsc.VectorSubcoreMesh(core_axis_name="c", subcore_axis_name="s")

    @functools.partial(
        pl.kernel, mesh=mesh,
        out_type=jax.ShapeDtypeStruct((_NC, n_dst_pad, HID), jnp.float32),
        scratch_types=[
            pltpu.VMEM((kc,), jnp.int32),
            pltpu.VMEM((kc,), jnp.int32),
            pltpu.VMEM((kc, HID), jnp.float32),
            pltpu.VMEM((kc, HID), jnp.float32),
            pltpu.VMEM((kc, HID), jnp.float32),
            pltpu.VMEM_SHARED((n_dst_pad, HID), jnp.float32),
            pltpu.SemaphoreType.DMA,
            pltpu.SemaphoreType.DMA,
            pltpu.SemaphoreType.DMA,
        ],
    )
    def k(a_hbm, b_hbm, c_hbm, si_hbm, di_hbm, out_hbm,
          sidx, didx, av, bv, cv, aggs, sem1, sem2, sem3):
        cc = lax.axis_index("c")
        ss = lax.axis_index("s")
        wid = cc * _NS + ss
        for r in range(16):
            for j in range(HID // _L):
                av[r, pl.ds(j * _L, _L)] = jnp.zeros((_L,), jnp.float32)
        row0 = ss * rows_per_tile

        @pl.loop(0, rows_per_tile, step=16)
        def _(r):
            pltpu.sync_copy(av.at[pl.ds(0, 16)], aggs.at[pl.ds(row0 + r, 16)])

        plsc.subcore_barrier()

        base_w = wid * per_w

        @pl.loop(0, n_chunks)
        def _(g):
            base = base_w + g * kc
            pltpu.sync_copy(si_hbm.at[pl.ds(base, kc)], sidx)
            pltpu.sync_copy(di_hbm.at[pl.ds(base, kc)], didx)
            cp_a = pltpu.async_copy(a_hbm.at[sidx], av, sem1)
            cp_b = pltpu.async_copy(b_hbm.at[didx], bv, sem2)
            cp_c = pltpu.async_copy(c_hbm.at[pl.ds(base, kc)], cv, sem3)
            cp_a.wait()
            cp_b.wait()
            cp_c.wait()

            @pl.loop(0, kc)
            def _(r):
                for j in range(HID // _L):
                    sl = pl.ds(j * _L, _L)
                    av[r, sl] = jnp.maximum(av[r, sl] + bv[r, sl] + cv[r, sl],
                                            0.0)

            pltpu.sync_copy(av, aggs.at[didx], add=True)

        plsc.subcore_barrier()
        pltpu.sync_copy(aggs.at[pl.ds(row0, rows_per_tile)],
                        out_hbm.at[cc, pl.ds(row0, rows_per_tile)])

    return k(a_tab, b_tab, c_tab, src_idx, dst_idx)


def _pad_rows(x, n):
    return jnp.pad(x, ((0, n - x.shape[0]), (0, 0)))


def _pad_edges(ei, ef, e_pad, garbage_dst):
    e = ei.shape[1]
    si = jnp.pad(ei[0].astype(jnp.int32), (0, e_pad - e))
    di = jnp.pad(ei[1].astype(jnp.int32), (0, e_pad - e),
                 constant_values=garbage_dst)
    ef = jnp.pad(ef, ((0, e_pad - e), (0, 0)))
    return si, di, ef


def _split_mp(p):
    we = p['W_e']
    wn = p['W_n']
    return (we[:HID], we[HID:2 * HID], we[2 * HID:], p['b_e'],
            wn[:HID], wn[HID:], p['b_n'])


def kernel(grid_features, g2m_edge_index, g2m_features, m2g_edge_index, m2g_features,
           m2m_edge_index_0, m2m_features_0, m2m_edge_index_1, m2m_features_1,
           up_edge_index_0, up_features_0, down_edge_index_0, down_features_0,
           mesh_features_0, mesh_features_1, params):
    p = params

    g2m_si, g2m_di, g2m_ef = _pad_edges(g2m_edge_index, g2m_features, 102400, N_MESH1)
    m2g_si, m2g_di, m2g_ef = _pad_edges(m2g_edge_index, m2g_features, 102400, N_MESH1)
    s1_si, s1_di, s1_ef = _pad_edges(m2m_edge_index_1, m2m_features_1, 204800, N_MESH1)
    s0_si, s0_di, s0_ef = _pad_edges(m2m_edge_index_0, m2m_features_0, 53248, N_MESH0)
    up_si, up_di, up_ef = _pad_edges(up_edge_index_0, up_features_0, 16384, N_MESH0)
    dn_si, dn_di, dn_ef = _pad_edges(down_edge_index_0, down_features_0, 16384, N_MESH0)

    (g2m_ws, g2m_wd, g2m_wf, g2m_be, g2m_w1, g2m_w2, g2m_bn) = _split_mp(p['g2m'])
    (m2g_ws, m2g_wd, m2g_wf, m2g_be, m2g_w1, m2g_w2, m2g_bn) = _split_mp(p['m2g'])
    (s1_ws, s1_wd, s1_wf, s1_be, s1_w1, s1_w2, s1_bn) = _split_mp(p['same_1'])
    (s0_ws, s0_wd, s0_wf, s0_be, s0_w1, s0_w2, s0_bn) = _split_mp(p['same_0'])
    (up_ws, up_wd, up_wf, up_be, up_w1, up_w2, up_bn) = _split_mp(p['up_0'])
    (dn_ws, dn_wd, dn_wf, dn_be, dn_w1, dn_w2, dn_bn) = _split_mp(p['down_0'])

    z128 = jnp.zeros((HID,), jnp.float32)

    c_g2m = _edge_c(g2m_ef, g2m_wf)
    c_m2g = _edge_c(m2g_ef, m2g_wf)
    c_s1 = _edge_c(s1_ef, s1_wf)
    c_s0 = _edge_c(s0_ef, s0_wf)
    c_up = _edge_c(up_ef, up_wf)
    c_dn = _edge_c(dn_ef, dn_wf)

    grid_rep = _mm(grid_features, p['W_grid_enc'], p['b_grid_enc'], 1000)
    mesh1 = _enc2(_pad_rows(mesh_features_1, N1P), p['W_mesh_enc_1'], p['b_mesh_enc_1'])
    mesh0 = _enc2(_pad_rows(mesh_features_0, N0P), p['W_mesh_enc_0'], p['b_mesh_enc_0'])

    a_t = _mm(grid_rep[:N1P], g2m_ws, z128, 1568)
    b_t = _mm(mesh1, g2m_wd, g2m_be, 1568)
    agg = _sc_edge_agg(a_t, b_t, c_g2m, g2m_si, g2m_di, N1P)
    mesh1 = _upd(mesh1, agg[0], agg[1], g2m_w1, g2m_w2, g2m_bn, 1568)

    a_t = _mm(mesh1, s1_ws, z128, 1568)
    b_t = _mm(mesh1, s1_wd, s1_be, 1568)
    agg = _sc_edge_agg(a_t, b_t, c_s1, s1_si, s1_di, N1P)
    mesh1 = _upd(mesh1, agg[0], agg[1], s1_w1, s1_w2, s1_bn, 1568)

    a_t = _mm(mesh1[:N0P], up_ws, z128, N0P)
    b_t = _mm(mesh0, up_wd, up_be, N0P)
    agg = _sc_edge_agg(a_t, b_t, c_up, up_si, up_di, N0P)
    mesh0 = _upd(mesh0, agg[0], agg[1], up_w1, up_w2, up_bn, N0P)

    a_t = _mm(mesh0, s0_ws, z128, N0P)
    b_t = _mm(mesh0, s0_wd, s0_be, N0P)
    agg = _sc_edge_agg(a_t, b_t, c_s0, s0_si, s0_di, N0P)
    mesh0 = _upd(mesh0, agg[0], agg[1], s0_w1, s0_w2, s0_bn, N0P)

    a_t = _mm(mesh0, dn_ws, z128, N0P)
    b_t = _mm(mesh1[:N0P], dn_wd, dn_be, N0P)
    agg = _sc_edge_agg(a_t, b_t, c_dn, dn_si, dn_di, N0P)
    agg0 = jnp.pad(agg[0, :N_MESH0], ((0, N1P - N_MESH0), (0, 0)))
    agg1 = jnp.pad(agg[1, :N_MESH0], ((0, N1P - N_MESH0), (0, 0)))
    mesh1 = _upd(mesh1, agg0, agg1, dn_w1, dn_w2, dn_bn, 1568)

    a_t = _mm(mesh1, s1_ws, z128, 1568)
    b_t = _mm(mesh1, s1_wd, s1_be, 1568)
    agg = _sc_edge_agg(a_t, b_t, c_s1, s1_si, s1_di, N1P)
    mesh1 = _upd(mesh1, agg[0], agg[1], s1_w1, s1_w2, s1_bn, 1568)

    a_t = _mm(mesh1, m2g_ws, z128, 1568)
    b_t = _mm(grid_rep[:N1P], m2g_wd, m2g_be, 1568)
    agg = _sc_edge_agg(a_t, b_t, c_m2g, m2g_si, m2g_di, N1P)
    agg0 = jnp.pad(agg[0, :N_MESH1], ((0, N_GRID - N_MESH1), (0, 0)))
    agg1 = jnp.pad(agg[1, :N_MESH1], ((0, N_GRID - N_MESH1), (0, 0)))
    delta = _m2g_dec(grid_rep, agg0, agg1, m2g_w1, m2g_w2, m2g_bn,
                     p['W_grid_dec'], p['b_grid_dec'], 1000)
    return delta


# speedup vs baseline: 1.8711x; 1.1495x over previous
import functools

import jax
import jax.numpy as jnp
from jax import lax
from jax.experimental import pallas as pl
from jax.experimental.pallas import tpu as pltpu
from jax.experimental.pallas import tpu_sc as plsc

N_GRID = 50000
N_MESH0 = 3125
N_MESH1 = 12500
HID = 128

N0P = 3328
N1P = 12544

_NC = 2
_NS = 16
_L = 16
_NW = _NC * _NS


def _mm_bias_kernel(x_ref, w_ref, b_ref, o_ref):
    o_ref[...] = jnp.dot(x_ref[...], w_ref[...],
                         preferred_element_type=jnp.float32) + b_ref[...]


def _mm(x, w, b, blk):
    n, kdim = x.shape
    return pl.pallas_call(
        _mm_bias_kernel,
        grid=(n // blk,),
        in_specs=[
            pl.BlockSpec((blk, kdim), lambda i: (i, 0)),
            pl.BlockSpec((kdim, w.shape[1]), lambda i: (0, 0)),
            pl.BlockSpec((w.shape[1],), lambda i: (0,)),
        ],
        out_specs=pl.BlockSpec((blk, w.shape[1]), lambda i: (i, 0)),
        out_shape=jax.ShapeDtypeStruct((n, w.shape[1]), jnp.float32),
    )(x, w, b)


def _enc2_kernel(x_ref, w_ref, b_ref, o_ref):
    x = x_ref[...]
    w = w_ref[...]
    o_ref[...] = x[:, 0:1] * w[0:1, :] + x[:, 1:2] * w[1:2, :] + b_ref[...]


def _enc2(x, w, b):
    n = x.shape[0]
    return pl.pallas_call(
        _enc2_kernel,
        grid=(1,),
        in_specs=[
            pl.BlockSpec((n, 2), lambda i: (0, 0)),
            pl.BlockSpec((2, HID), lambda i: (0, 0)),
            pl.BlockSpec((HID,), lambda i: (0,)),
        ],
        out_specs=pl.BlockSpec((n, HID), lambda i: (0, 0)),
        out_shape=jax.ShapeDtypeStruct((n, HID), jnp.float32),
    )(x, w, b)


def _c_kernel(ef_ref, w_ref, o_ref):
    ef = ef_ref[...]
    w = w_ref[...]
    o_ref[...] = (ef[:, 0:1] * w[0:1, :] + ef[:, 1:2] * w[1:2, :]
                  + ef[:, 2:3] * w[2:3, :] + ef[:, 3:4] * w[3:4, :])


def _edge_c(ef, w, blk=2048):
    n = ef.shape[0]
    return pl.pallas_call(
        _c_kernel,
        grid=(n // blk,),
        in_specs=[
            pl.BlockSpec((blk, 4), lambda i: (i, 0)),
            pl.BlockSpec((4, HID), lambda i: (0, 0)),
        ],
        out_specs=pl.BlockSpec((blk, HID), lambda i: (i, 0)),
        out_shape=jax.ShapeDtypeStruct((n, HID), jnp.float32),
    )(ef, w)


def _upd_kernel(x_ref, a0_ref, a1_ref, w1_ref, w2_ref, b_ref, o_ref):
    x = x_ref[...]
    agg = a0_ref[...] + a1_ref[...]
    o_ref[...] = (x + jnp.dot(x, w1_ref[...], preferred_element_type=jnp.float32)
                  + jnp.dot(agg, w2_ref[...], preferred_element_type=jnp.float32)
                  + b_ref[...])


def _upd(x, a0, a1, w1, w2, b, blk):
    n = x.shape[0]
    return pl.pallas_call(
        _upd_kernel,
        grid=(n // blk,),
        in_specs=[
            pl.BlockSpec((blk, HID), lambda i: (i, 0)),
            pl.BlockSpec((blk, HID), lambda i: (i, 0)),
            pl.BlockSpec((blk, HID), lambda i: (i, 0)),
            pl.BlockSpec((HID, HID), lambda i: (0, 0)),
            pl.BlockSpec((HID, HID), lambda i: (0, 0)),
            pl.BlockSpec((HID,), lambda i: (0,)),
        ],
        out_specs=pl.BlockSpec((blk, HID), lambda i: (i, 0)),
        out_shape=jax.ShapeDtypeStruct((n, HID), jnp.float32),
    )(x, a0, a1, w1, w2, b)


def _m2g_dec_kernel(x_ref, a0_ref, a1_ref, w1_ref, w2_ref, bn_ref,
                    wd_ref, bd_ref, o_ref):
    x = x_ref[...]
    agg = a0_ref[...] + a1_ref[...]
    t = (x + jnp.dot(x, w1_ref[...], preferred_element_type=jnp.float32)
         + jnp.dot(agg, w2_ref[...], preferred_element_type=jnp.float32)
         + bn_ref[...])
    o_ref[...] = jnp.dot(t, wd_ref[...],
                         preferred_element_type=jnp.float32) + bd_ref[...]


def _m2g_dec(x, a0, a1, w1, w2, bn, wd, bd, blk=1000):
    n = x.shape[0]
    return pl.pallas_call(
        _m2g_dec_kernel,
        grid=(n // blk,),
        in_specs=[
            pl.BlockSpec((blk, HID), lambda i: (i, 0)),
            pl.BlockSpec((blk, HID), lambda i: (i, 0)),
            pl.BlockSpec((blk, HID), lambda i: (i, 0)),
            pl.BlockSpec((HID, HID), lambda i: (0, 0)),
            pl.BlockSpec((HID, HID), lambda i: (0, 0)),
            pl.BlockSpec((HID,), lambda i: (0,)),
            pl.BlockSpec((HID, HID), lambda i: (0, 0)),
            pl.BlockSpec((HID,), lambda i: (0,)),
        ],
        out_specs=pl.BlockSpec((blk, HID), lambda i: (i, 0)),
        out_shape=jax.ShapeDtypeStruct((n, HID), jnp.float32),
    )(x, a0, a1, w1, w2, bn, wd, bd)


def _sc_edge_agg(a_tab, b_tab, c_tab, src_idx, dst_idx, n_dst_pad):
    e_pad = src_idx.shape[0]
    per_w = e_pad // _NW
    budget = 2_097_151 - n_dst_pad * HID
    kc = 128
    while kc > 8 and (2 * kc * (3 * HID + 2) * _NS > budget or per_w % kc
                      or (per_w // kc) % 2):
        kc //= 2
    n_chunks = per_w // kc
    rows_per_tile = n_dst_pad // _NS
    mesh = plsc.VectorSubcoreMesh(core_axis_name="c", subcore_axis_name="s")

    @functools.partial(
        pl.kernel, mesh=mesh,
        out_type=jax.ShapeDtypeStruct((_NC, n_dst_pad, HID), jnp.float32),
        scratch_types=[
            pltpu.VMEM((2, kc), jnp.int32),
            pltpu.VMEM((2, kc), jnp.int32),
            pltpu.VMEM((2, kc, HID), jnp.float32),
            pltpu.VMEM((2, kc, HID), jnp.float32),
            pltpu.VMEM((2, kc, HID), jnp.float32),
            pltpu.VMEM_SHARED((n_dst_pad, HID), jnp.float32),
            pltpu.SemaphoreType.DMA,
            pltpu.SemaphoreType.DMA,
        ],
    )
    def k(a_hbm, b_hbm, c_hbm, si_hbm, di_hbm, out_hbm,
          sidx, didx, av, bv, cv, aggs, sem0, sem1):
        cc = lax.axis_index("c")
        ss = lax.axis_index("s")
        wid = cc * _NS + ss
        for r in range(16):
            for j in range(HID // _L):
                av[0, r, pl.ds(j * _L, _L)] = jnp.zeros((_L,), jnp.float32)
        row0 = ss * rows_per_tile

        @pl.loop(0, rows_per_tile, step=16)
        def _(r):
            pltpu.sync_copy(av.at[0, pl.ds(0, 16)],
                            aggs.at[pl.ds(row0 + r, 16)])

        plsc.subcore_barrier()

        base_w = wid * per_w
        sems = (sem0, sem1)

        def start(g, b):
            base = base_w + g * kc
            pltpu.sync_copy(si_hbm.at[pl.ds(base, kc)], sidx.at[b])
            pltpu.sync_copy(di_hbm.at[pl.ds(base, kc)], didx.at[b])
            pltpu.async_copy(a_hbm.at[sidx.at[b]], av.at[b], sems[b])
            pltpu.async_copy(b_hbm.at[didx.at[b]], bv.at[b], sems[b])
            pltpu.async_copy(c_hbm.at[pl.ds(base, kc)], cv.at[b], sems[b])

        def finish(b):
            pltpu.make_async_copy(a_hbm.at[sidx.at[b]], av.at[b],
                                  sems[b]).wait()
            pltpu.make_async_copy(b_hbm.at[didx.at[b]], bv.at[b],
                                  sems[b]).wait()
            pltpu.make_async_copy(c_hbm.at[pl.ds(0, kc)], cv.at[b],
                                  sems[b]).wait()

            @pl.loop(0, kc)
            def _(r):
                for j in range(HID // _L):
                    sl = pl.ds(j * _L, _L)
                    av[b, r, sl] = jnp.maximum(
                        av[b, r, sl] + bv[b, r, sl] + cv[b, r, sl], 0.0)

            pltpu.sync_copy(av.at[b], aggs.at[didx.at[b]], add=True)

        start(0, 0)

        @pl.loop(0, n_chunks, step=2)
        def _(g0):
            start(g0 + 1, 1)
            finish(0)

            @pl.when(g0 + 2 < n_chunks)
            def _():
                start(g0 + 2, 0)

            finish(1)

        plsc.subcore_barrier()
        pltpu.sync_copy(aggs.at[pl.ds(row0, rows_per_tile)],
                        out_hbm.at[cc, pl.ds(row0, rows_per_tile)])

    return k(a_tab, b_tab, c_tab, src_idx, dst_idx)


def _pad_rows(x, n):
    return jnp.pad(x, ((0, n - x.shape[0]), (0, 0)))


def _pad_edges(ei, ef, e_pad, garbage_dst):
    e = ei.shape[1]
    si = jnp.pad(ei[0].astype(jnp.int32), (0, e_pad - e))
    di = jnp.pad(ei[1].astype(jnp.int32), (0, e_pad - e),
                 constant_values=garbage_dst)
    ef = jnp.pad(ef, ((0, e_pad - e), (0, 0)))
    return si, di, ef


def _split_mp(p):
    we = p['W_e']
    wn = p['W_n']
    return (we[:HID], we[HID:2 * HID], we[2 * HID:], p['b_e'],
            wn[:HID], wn[HID:], p['b_n'])


def kernel(grid_features, g2m_edge_index, g2m_features, m2g_edge_index, m2g_features,
           m2m_edge_index_0, m2m_features_0, m2m_edge_index_1, m2m_features_1,
           up_edge_index_0, up_features_0, down_edge_index_0, down_features_0,
           mesh_features_0, mesh_features_1, params):
    p = params

    g2m_si, g2m_di, g2m_ef = _pad_edges(g2m_edge_index, g2m_features, 102400, N_MESH1)
    m2g_si, m2g_di, m2g_ef = _pad_edges(m2g_edge_index, m2g_features, 102400, N_MESH1)
    s1_si, s1_di, s1_ef = _pad_edges(m2m_edge_index_1, m2m_features_1, 204800, N_MESH1)
    s0_si, s0_di, s0_ef = _pad_edges(m2m_edge_index_0, m2m_features_0, 53248, N_MESH0)
    up_si, up_di, up_ef = _pad_edges(up_edge_index_0, up_features_0, 16384, N_MESH0)
    dn_si, dn_di, dn_ef = _pad_edges(down_edge_index_0, down_features_0, 16384, N_MESH0)

    (g2m_ws, g2m_wd, g2m_wf, g2m_be, g2m_w1, g2m_w2, g2m_bn) = _split_mp(p['g2m'])
    (m2g_ws, m2g_wd, m2g_wf, m2g_be, m2g_w1, m2g_w2, m2g_bn) = _split_mp(p['m2g'])
    (s1_ws, s1_wd, s1_wf, s1_be, s1_w1, s1_w2, s1_bn) = _split_mp(p['same_1'])
    (s0_ws, s0_wd, s0_wf, s0_be, s0_w1, s0_w2, s0_bn) = _split_mp(p['same_0'])
    (up_ws, up_wd, up_wf, up_be, up_w1, up_w2, up_bn) = _split_mp(p['up_0'])
    (dn_ws, dn_wd, dn_wf, dn_be, dn_w1, dn_w2, dn_bn) = _split_mp(p['down_0'])

    z128 = jnp.zeros((HID,), jnp.float32)

    c_g2m = _edge_c(g2m_ef, g2m_wf)
    c_m2g = _edge_c(m2g_ef, m2g_wf)
    c_s1 = _edge_c(s1_ef, s1_wf)
    c_s0 = _edge_c(s0_ef, s0_wf)
    c_up = _edge_c(up_ef, up_wf)
    c_dn = _edge_c(dn_ef, dn_wf)

    grid_rep = _mm(grid_features, p['W_grid_enc'], p['b_grid_enc'], 1000)
    mesh1 = _enc2(_pad_rows(mesh_features_1, N1P), p['W_mesh_enc_1'], p['b_mesh_enc_1'])
    mesh0 = _enc2(_pad_rows(mesh_features_0, N0P), p['W_mesh_enc_0'], p['b_mesh_enc_0'])

    a_t = _mm(grid_rep[:N1P], g2m_ws, z128, 1568)
    b_t = _mm(mesh1, g2m_wd, g2m_be, 1568)
    agg = _sc_edge_agg(a_t, b_t, c_g2m, g2m_si, g2m_di, N1P)
    mesh1 = _upd(mesh1, agg[0], agg[1], g2m_w1, g2m_w2, g2m_bn, 1568)

    a_t = _mm(mesh1, s1_ws, z128, 1568)
    b_t = _mm(mesh1, s1_wd, s1_be, 1568)
    agg = _sc_edge_agg(a_t, b_t, c_s1, s1_si, s1_di, N1P)
    mesh1 = _upd(mesh1, agg[0], agg[1], s1_w1, s1_w2, s1_bn, 1568)

    a_t = _mm(mesh1[:N0P], up_ws, z128, N0P)
    b_t = _mm(mesh0, up_wd, up_be, N0P)
    agg = _sc_edge_agg(a_t, b_t, c_up, up_si, up_di, N0P)
    mesh0 = _upd(mesh0, agg[0], agg[1], up_w1, up_w2, up_bn, N0P)

    a_t = _mm(mesh0, s0_ws, z128, N0P)
    b_t = _mm(mesh0, s0_wd, s0_be, N0P)
    agg = _sc_edge_agg(a_t, b_t, c_s0, s0_si, s0_di, N0P)
    mesh0 = _upd(mesh0, agg[0], agg[1], s0_w1, s0_w2, s0_bn, N0P)

    a_t = _mm(mesh0, dn_ws, z128, N0P)
    b_t = _mm(mesh1[:N0P], dn_wd, dn_be, N0P)
    agg = _sc_edge_agg(a_t, b_t, c_dn, dn_si, dn_di, N0P)
    agg0 = jnp.pad(agg[0, :N_MESH0], ((0, N1P - N_MESH0), (0, 0)))
    agg1 = jnp.pad(agg[1, :N_MESH0], ((0, N1P - N_MESH0), (0, 0)))
    mesh1 = _upd(mesh1, agg0, agg1, dn_w1, dn_w2, dn_bn, 1568)

    a_t = _mm(mesh1, s1_ws, z128, 1568)
    b_t = _mm(mesh1, s1_wd, s1_be, 1568)
    agg = _sc_edge_agg(a_t, b_t, c_s1, s1_si, s1_di, N1P)
    mesh1 = _upd(mesh1, agg[0], agg[1], s1_w1, s1_w2, s1_bn, 1568)

    a_t = _mm(mesh1, m2g_ws, z128, 1568)
    b_t = _mm(grid_rep[:N1P], m2g_wd, m2g_be, 1568)
    agg = _sc_edge_agg(a_t, b_t, c_m2g, m2g_si, m2g_di, N1P)
    agg0 = jnp.pad(agg[0, :N_MESH1], ((0, N_GRID - N_MESH1), (0, 0)))
    agg1 = jnp.pad(agg[1, :N_MESH1], ((0, N_GRID - N_MESH1), (0, 0)))
    delta = _m2g_dec(grid_rep, agg0, agg1, m2g_w1, m2g_w2, m2g_bn,
                     p['W_grid_dec'], p['b_grid_dec'], 1000)
    return delta
